# baseline (speedup 1.0000x reference)
import jax
import jax.numpy as jnp
from jax.experimental import pallas as pl
from jax.experimental.pallas import tpu as pltpu

_N = 10000
_ROWS = 200
_NB = _N // (2 * _ROWS)


def _proj_body(feat_ref, w1_ref, u_ref):
    u_ref[...] = jnp.dot(feat_ref[...], w1_ref[...],
                         preferred_element_type=jnp.float32)


def _body(u_ref, adja_ref, adjb_ref, b1_ref, w2_ref, b2_ref,
          x1a_ref, x1b_ref, outa_ref, outb_ref, v_ref):
    i = pl.program_id(0)

    def _softmax(h):
        m = jnp.max(h, axis=1, keepdims=True)
        e = jnp.exp(h - m)
        s = jnp.sum(e, axis=1, keepdims=True)
        return h - m - jnp.log(s)

    @pl.when(i < _NB)
    def _():
        u = u_ref[...]
        w2 = w2_ref[...]
        ha = jnp.dot(adja_ref[...], u, preferred_element_type=jnp.float32)
        x1a = jnp.maximum(ha + b1_ref[...], 0.0)
        x1a_ref[...] = x1a
        v_ref[pl.ds((2 * i) * _ROWS, _ROWS), :] = jnp.dot(
            x1a, w2, preferred_element_type=jnp.float32)
        hb = jnp.dot(adjb_ref[...], u, preferred_element_type=jnp.float32)
        x1b = jnp.maximum(hb + b1_ref[...], 0.0)
        x1b_ref[...] = x1b
        v_ref[pl.ds((2 * i + 1) * _ROWS, _ROWS), :] = jnp.dot(
            x1b, w2, preferred_element_type=jnp.float32)

    @pl.when(i >= _NB)
    def _():
        v = v_ref[...]
        ha = jnp.dot(adja_ref[...], v, preferred_element_type=jnp.float32)
        outa_ref[...] = _softmax(ha + b2_ref[...])
        hb = jnp.dot(adjb_ref[...], v, preferred_element_type=jnp.float32)
        outb_ref[...] = _softmax(hb + b2_ref[...])


def kernel(feature, adj, W1, b1, W2, b2):
    f_in = feature.shape[1]
    hid = W1.shape[1]
    dim = W2.shape[1]
    b1r = b1.reshape(1, hid)
    b2r = b2.reshape(1, dim)

    u = pl.pallas_call(
        _proj_body,
        in_specs=[
            pl.BlockSpec((_N, f_in), lambda: (0, 0)),
            pl.BlockSpec((f_in, hid), lambda: (0, 0)),
        ],
        out_specs=pl.BlockSpec((_N, hid), lambda: (0, 0)),
        out_shape=jax.ShapeDtypeStruct((_N, hid), jnp.float32),
    )(feature, W1)

    def _stream_idx(off):
        return lambda i: (jnp.where(i < _NB, 2 * i + off,
                                    2 * (i - _NB) + off), 0)

    x1a, x1b, outa, outb = pl.pallas_call(
        _body,
        grid=(2 * _NB,),
        in_specs=[
            pl.BlockSpec((_N, hid), lambda i: (0, 0)),
            pl.BlockSpec((_ROWS, _N), _stream_idx(0)),
            pl.BlockSpec((_ROWS, _N), _stream_idx(1)),
            pl.BlockSpec((1, hid), lambda i: (0, 0)),
            pl.BlockSpec((hid, dim), lambda i: (0, 0)),
            pl.BlockSpec((1, dim), lambda i: (0, 0)),
        ],
        out_specs=[
            pl.BlockSpec((_ROWS, hid),
                         lambda i: (jnp.where(i < _NB, i, _NB - 1), 0)),
            pl.BlockSpec((_ROWS, hid),
                         lambda i: (jnp.where(i < _NB, i, _NB - 1), 0)),
            pl.BlockSpec((_ROWS, dim),
                         lambda i: (jnp.where(i < _NB, 0, i - _NB), 0)),
            pl.BlockSpec((_ROWS, dim),
                         lambda i: (jnp.where(i < _NB, 0, i - _NB), 0)),
        ],
        out_shape=[
            jax.ShapeDtypeStruct((_N // 2, hid), jnp.float32),
            jax.ShapeDtypeStruct((_N // 2, hid), jnp.float32),
            jax.ShapeDtypeStruct((_N // 2, dim), jnp.float32),
            jax.ShapeDtypeStruct((_N // 2, dim), jnp.float32),
        ],
        scratch_shapes=[
            pltpu.VMEM((_N, dim), jnp.float32),
        ],
    )(u, adj, adj, b1r, W2, b2r)

    x1 = jnp.stack([x1a.reshape(_NB, _ROWS, hid),
                    x1b.reshape(_NB, _ROWS, hid)], axis=1).reshape(_N, hid)
    out = jnp.stack([outa.reshape(_NB, _ROWS, dim),
                     outb.reshape(_NB, _ROWS, dim)], axis=1).reshape(_N, dim)
    return (x1, out)


# speedup vs baseline: 1.0942x; 1.0942x over previous
import jax
import jax.numpy as jnp
from jax.experimental import pallas as pl
from jax.experimental.pallas import tpu as pltpu

_N = 10000
_ROWS = 400


def _layer1_body(feat_ref, adj_ref, w1_ref, b1_ref, x1_ref, u_ref):
    @pl.when(pl.program_id(0) == 0)
    def _():
        u = jnp.dot(feat_ref[...], w1_ref[...],
                    preferred_element_type=jnp.float32)
        u_ref[...] = u.astype(jnp.bfloat16)

    a = adj_ref[...].astype(jnp.bfloat16)
    h = jnp.dot(a, u_ref[...], preferred_element_type=jnp.float32)
    x1_ref[...] = jnp.maximum(h + b1_ref[...], 0.0)


def _layer2_body(x1_ref, adj_ref, w2_ref, b2_ref, out_ref, v_ref):
    @pl.when(pl.program_id(0) == 0)
    def _():
        v = jnp.dot(x1_ref[...], w2_ref[...],
                    preferred_element_type=jnp.float32)
        v_ref[...] = v.astype(jnp.bfloat16)

    a = adj_ref[...].astype(jnp.bfloat16)
    h = jnp.dot(a, v_ref[...], preferred_element_type=jnp.float32)
    h = h + b2_ref[...]
    m = jnp.max(h, axis=1, keepdims=True)
    e = jnp.exp(h - m)
    s = jnp.sum(e, axis=1, keepdims=True)
    out_ref[...] = h - m - jnp.log(s)


def kernel(feature, adj, W1, b1, W2, b2):
    f_in = feature.shape[1]
    hid = W1.shape[1]
    dim = W2.shape[1]
    nsteps = _N // _ROWS
    b1r = b1.reshape(1, hid)
    b2r = b2.reshape(1, dim)

    x1 = pl.pallas_call(
        _layer1_body,
        grid=(nsteps,),
        in_specs=[
            pl.BlockSpec((_N, f_in), lambda i: (0, 0)),
            pl.BlockSpec((_ROWS, _N), lambda i: (i, 0)),
            pl.BlockSpec((f_in, hid), lambda i: (0, 0)),
            pl.BlockSpec((1, hid), lambda i: (0, 0)),
        ],
        out_specs=pl.BlockSpec((_ROWS, hid), lambda i: (i, 0)),
        out_shape=jax.ShapeDtypeStruct((_N, hid), jnp.float32),
        scratch_shapes=[pltpu.VMEM((_N, hid), jnp.bfloat16)],
    )(feature, adj, W1, b1r)

    out = pl.pallas_call(
        _layer2_body,
        grid=(nsteps,),
        in_specs=[
            pl.BlockSpec((_N, hid), lambda i: (0, 0)),
            pl.BlockSpec((_ROWS, _N), lambda i: (i, 0)),
            pl.BlockSpec((hid, dim), lambda i: (0, 0)),
            pl.BlockSpec((1, dim), lambda i: (0, 0)),
        ],
        out_specs=pl.BlockSpec((_ROWS, dim), lambda i: (i, 0)),
        out_shape=jax.ShapeDtypeStruct((_N, dim), jnp.float32),
        scratch_shapes=[pltpu.VMEM((_N, dim), jnp.bfloat16)],
    )(x1, adj, W2, b2r)

    return (x1, out)
